# baseline (speedup 1.0000x reference)
import functools

import jax
import jax.numpy as jnp
from jax import lax
from jax.experimental import pallas as pl
from jax.experimental.pallas import tpu as pltpu
from jax.experimental.pallas import tpu_sc as plsc

_B, _T, _D, _K = 16, 2048, 256, 1024
_N = _B * _T
_RB = 1024
_NB = _N // _RB

_NC, _NS = 2, 16
_NW = _NC * _NS
_B_PER_W = _N // _NW
_CH = 128
_NCHUNK = _B_PER_W // _CH


def _tc_body(z_ref, c2_ref, m_ref, cbm2_ref, idx_ref, pc_ref, pn_ref):
    zb = z_ref[...]
    cbm2 = cbm2_ref[...]
    dots = lax.dot_general(cbm2, zb, (((1,), (1,)), ((), ())))
    c2 = c2_ref[...]
    z2col = jnp.sum(zb * zb, axis=1, keepdims=True)
    z2 = z2col.T
    dist = (z2 + dots) + c2
    minv = jnp.min(dist, axis=0, keepdims=True)
    kio = lax.broadcasted_iota(jnp.int32, dist.shape, 0)
    idx = jnp.min(jnp.where(dist == minv, kio, _K), axis=0)
    mrow = m_ref[0, 0, :]
    commit_p = jnp.sum(minv[0] * mrow)
    cnt_p = jnp.sum(mrow)
    idx_ref[0, 0, :] = idx
    pc_ref[0, 0, :] = jnp.full((128,), commit_p, jnp.float32)
    pn_ref[0, 0, :] = jnp.full((128,), cnt_p, jnp.float32)


def _tc_argmin(zf, c2col, maskf, cbm2):
    nb = zf.shape[0] // _RB
    return pl.pallas_call(
        _tc_body,
        grid=(nb,),
        in_specs=[
            pl.BlockSpec((_RB, _D), lambda i: (i, 0)),
            pl.BlockSpec((_K, 1), lambda i: (0, 0)),
            pl.BlockSpec((1, 1, _RB), lambda i: (i, 0, 0)),
            pl.BlockSpec((_K, _D), lambda i: (0, 0)),
        ],
        out_specs=[
            pl.BlockSpec((1, 1, _RB), lambda i: (i, 0, 0)),
            pl.BlockSpec((1, 1, 128), lambda i: (i, 0, 0)),
            pl.BlockSpec((1, 1, 128), lambda i: (i, 0, 0)),
        ],
        out_shape=[
            jax.ShapeDtypeStruct((nb, 1, _RB), jnp.int32),
            jax.ShapeDtypeStruct((nb, 1, 128), jnp.float32),
            jax.ShapeDtypeStruct((nb, 1, 128), jnp.float32),
        ],
        compiler_params=pltpu.CompilerParams(
            dimension_semantics=("arbitrary",),
        ),
    )(zf, c2col, maskf, cbm2)


def _make_sc_gather_body(bpw, nchunk):
    def _sc_gather_body(cb_hbm, idx_hbm, out_hbm, idx_v, rows_v0, rows_v1,
                        sem0, sem1):
        wid = lax.axis_index("s") * _NC + lax.axis_index("c")
        base = wid * bpw
        rows_v = (rows_v0, rows_v1)
        sem = (sem0, sem1)
        pltpu.sync_copy(idx_hbm.at[pl.ds(base, bpw)], idx_v)
        descs = [None, None]
        descs[0] = pltpu.async_copy(cb_hbm.at[idx_v.at[pl.ds(0, _CH)]],
                                    rows_v[0], sem[0])
        for c in range(nchunk):
            cur, nxt = c % 2, (c + 1) % 2
            if c + 1 < nchunk:
                descs[nxt] = pltpu.async_copy(
                    cb_hbm.at[idx_v.at[pl.ds((c + 1) * _CH, _CH)]],
                    rows_v[nxt], sem[nxt])
            descs[cur].wait()
            pltpu.sync_copy(rows_v[cur], out_hbm.at[pl.ds(base + c * _CH,
                                                          _CH)])
    return _sc_gather_body


@functools.lru_cache(maxsize=4)
def _sc_gather_kernel(nrows):
    bpw = nrows // _NW
    return pl.kernel(
        _make_sc_gather_body(bpw, bpw // _CH),
        out_type=jax.ShapeDtypeStruct((nrows, _D), jnp.float32),
        mesh=plsc.VectorSubcoreMesh(core_axis_name="c", subcore_axis_name="s",
                                    num_cores=_NC, num_subcores=_NS),
        scratch_types=[
            pltpu.VMEM((bpw,), jnp.int32),
            pltpu.VMEM((_CH, _D), jnp.float32),
            pltpu.VMEM((_CH, _D), jnp.float32),
            pltpu.SemaphoreType.DMA,
            pltpu.SemaphoreType.DMA,
        ],
    )


_SPLIT = 2
_HALF = _N // _SPLIT


def kernel(z, mask, codebook):
    zf = z.reshape(_N, _D)
    c2col = jnp.sum(codebook * codebook, axis=-1).reshape(_K, 1)
    cbm2 = codebook * -2.0
    maskf = mask.astype(z.dtype).reshape(_NB, 1, _RB)
    nbh = _NB // _SPLIT
    idxs, qs, pcs, pns = [], [], [], []
    for p in range(_SPLIT):
        idx3, pc, pn = _tc_argmin(zf[p * _HALF:(p + 1) * _HALF], c2col,
                                  maskf[p * nbh:(p + 1) * nbh], cbm2)
        idxs.append(idx3.reshape(_HALF))
        pcs.append(pc)
        pns.append(pn)
        qs.append(_sc_gather_kernel(_HALF)(codebook, idxs[-1]))
    idx = jnp.concatenate(idxs)
    quantized = jnp.concatenate(qs)
    pc = jnp.concatenate(pcs)
    pn = jnp.concatenate(pns)
    cnt = jnp.sum(pn[:, 0, 0])
    denom = jnp.maximum(cnt, 1.0) * _D
    commit_loss = jnp.sum(pc[:, 0, 0]) / denom
    return quantized.reshape(_B, _T, _D), idx.reshape(_B, _T), commit_loss


# speedup vs baseline: 1.3478x; 1.3478x over previous
import functools

import jax
import jax.numpy as jnp
from jax import lax
from jax.experimental import pallas as pl
from jax.experimental.pallas import tpu as pltpu
from jax.experimental.pallas import tpu_sc as plsc

_B, _T, _D, _K = 16, 2048, 256, 1024
_N = _B * _T
_RB = 2048
_NB = _N // _RB

_NC, _NS = 2, 16
_NW = _NC * _NS
_B_PER_W = _N // _NW
_CH = 128
_NCHUNK = _B_PER_W // _CH


def _tc_body(z_ref, c2_ref, m_ref, cbm2_ref, idx_ref, pc_ref, pn_ref):
    zb = z_ref[...]
    cbm2 = cbm2_ref[...]
    dots = lax.dot_general(cbm2, zb, (((1,), (1,)), ((), ())))
    c2 = c2_ref[...]
    z2col = jnp.sum(zb * zb, axis=1, keepdims=True)
    z2 = z2col.T
    dist = (z2 + dots) + c2
    minv = jnp.min(dist, axis=0, keepdims=True)
    kio = lax.broadcasted_iota(jnp.int32, dist.shape, 0)
    idx = jnp.min(jnp.where(dist == minv, kio, _K), axis=0)
    mrow = m_ref[0, 0, :]
    commit_p = jnp.sum(minv[0] * mrow)
    cnt_p = jnp.sum(mrow)
    idx_ref[0, 0, :] = idx
    pc_ref[0, 0, :] = jnp.full((128,), commit_p, jnp.float32)
    pn_ref[0, 0, :] = jnp.full((128,), cnt_p, jnp.float32)


def _tc_argmin(zf, c2col, maskf, cbm2):
    return pl.pallas_call(
        _tc_body,
        grid=(_NB,),
        in_specs=[
            pl.BlockSpec((_RB, _D), lambda i: (i, 0)),
            pl.BlockSpec((_K, 1), lambda i: (0, 0)),
            pl.BlockSpec((1, 1, _RB), lambda i: (i, 0, 0)),
            pl.BlockSpec((_K, _D), lambda i: (0, 0)),
        ],
        out_specs=[
            pl.BlockSpec((1, 1, _RB), lambda i: (i, 0, 0)),
            pl.BlockSpec((1, 1, 128), lambda i: (i, 0, 0)),
            pl.BlockSpec((1, 1, 128), lambda i: (i, 0, 0)),
        ],
        out_shape=[
            jax.ShapeDtypeStruct((_NB, 1, _RB), jnp.int32),
            jax.ShapeDtypeStruct((_NB, 1, 128), jnp.float32),
            jax.ShapeDtypeStruct((_NB, 1, 128), jnp.float32),
        ],
        compiler_params=pltpu.CompilerParams(
            dimension_semantics=("arbitrary",),
        ),
    )(zf, c2col, maskf, cbm2)


def _sc_gather_body(cb_hbm, idx_hbm, out_hbm, idx_v, rows_v0, rows_v1,
                    sem0, sem1):
    wid = lax.axis_index("s") * _NC + lax.axis_index("c")
    base = wid * _B_PER_W
    rows_v = (rows_v0, rows_v1)
    sem = (sem0, sem1)
    pltpu.sync_copy(idx_hbm.at[pl.ds(base, _B_PER_W)], idx_v)
    descs = [None, None]
    descs[0] = pltpu.async_copy(cb_hbm.at[idx_v.at[pl.ds(0, _CH)]], rows_v[0],
                                sem[0])
    for c in range(_NCHUNK):
        cur, nxt = c % 2, (c + 1) % 2
        if c + 1 < _NCHUNK:
            descs[nxt] = pltpu.async_copy(
                cb_hbm.at[idx_v.at[pl.ds((c + 1) * _CH, _CH)]], rows_v[nxt],
                sem[nxt])
        descs[cur].wait()
        pltpu.sync_copy(rows_v[cur], out_hbm.at[pl.ds(base + c * _CH, _CH)])


@functools.lru_cache(maxsize=1)
def _sc_gather_kernel():
    return pl.kernel(
        _sc_gather_body,
        out_type=jax.ShapeDtypeStruct((_N, _D), jnp.float32),
        mesh=plsc.VectorSubcoreMesh(core_axis_name="c", subcore_axis_name="s",
                                    num_cores=_NC, num_subcores=_NS),
        scratch_types=[
            pltpu.VMEM((_B_PER_W,), jnp.int32),
            pltpu.VMEM((_CH, _D), jnp.float32),
            pltpu.VMEM((_CH, _D), jnp.float32),
            pltpu.SemaphoreType.DMA,
            pltpu.SemaphoreType.DMA,
        ],
    )


def kernel(z, mask, codebook):
    zf = z.reshape(_N, _D)
    c2col = jnp.sum(codebook * codebook, axis=-1).reshape(_K, 1)
    cbm2 = codebook * -2.0
    maskf = mask.astype(z.dtype).reshape(_NB, 1, _RB)
    idx3, pc, pn = _tc_argmin(zf, c2col, maskf, cbm2)
    idx = idx3.reshape(_N)
    quantized = _sc_gather_kernel()(codebook, idx)
    cnt = jnp.sum(pn[:, 0, 0])
    denom = jnp.maximum(cnt, 1.0) * _D
    commit_loss = jnp.sum(pc[:, 0, 0]) / denom
    return quantized.reshape(_B, _T, _D), idx.reshape(_B, _T), commit_loss


# speedup vs baseline: 1.4342x; 1.0641x over previous
import functools

import jax
import jax.numpy as jnp
from jax import lax
from jax.experimental import pallas as pl
from jax.experimental.pallas import tpu as pltpu
from jax.experimental.pallas import tpu_sc as plsc

_B, _T, _D, _K = 16, 2048, 256, 1024
_N = _B * _T
_RB = 1024
_NB = _N // _RB

_NC, _NS = 2, 16
_NW = _NC * _NS
_B_PER_W = _N // _NW
_CH = 128
_NCHUNK = _B_PER_W // _CH


def _tc_body(z_ref, c2_ref, m_ref, cbm2_ref, idx_ref, pc_ref, pn_ref):
    zb = z_ref[...]
    cbm2 = cbm2_ref[...]
    dots = lax.dot_general(cbm2, zb, (((1,), (1,)), ((), ())))
    c2 = c2_ref[...]
    z2col = jnp.sum(zb * zb, axis=1, keepdims=True)
    z2 = z2col.T
    dist = (z2 + dots) + c2
    minv = jnp.min(dist, axis=0, keepdims=True)
    kio = lax.broadcasted_iota(jnp.int32, (_K, 1), 0).astype(jnp.float32)
    idxf = jnp.min(jnp.where(dist == minv, kio, float(_K)), axis=0)
    idx = idxf.astype(jnp.int32)
    mrow = m_ref[0, 0, :]
    commit_p = jnp.sum(minv[0] * mrow)
    cnt_p = jnp.sum(mrow)
    idx_ref[0, 0, :] = idx
    pc_ref[0, 0, :] = jnp.full((128,), commit_p, jnp.float32)
    pn_ref[0, 0, :] = jnp.full((128,), cnt_p, jnp.float32)


def _tc_argmin(zf, c2col, maskf, cbm2):
    return pl.pallas_call(
        _tc_body,
        grid=(_NB,),
        in_specs=[
            pl.BlockSpec((_RB, _D), lambda i: (i, 0)),
            pl.BlockSpec((_K, 1), lambda i: (0, 0)),
            pl.BlockSpec((1, 1, _RB), lambda i: (i, 0, 0)),
            pl.BlockSpec((_K, _D), lambda i: (0, 0)),
        ],
        out_specs=[
            pl.BlockSpec((1, 1, _RB), lambda i: (i, 0, 0)),
            pl.BlockSpec((1, 1, 128), lambda i: (i, 0, 0)),
            pl.BlockSpec((1, 1, 128), lambda i: (i, 0, 0)),
        ],
        out_shape=[
            jax.ShapeDtypeStruct((_NB, 1, _RB), jnp.int32),
            jax.ShapeDtypeStruct((_NB, 1, 128), jnp.float32),
            jax.ShapeDtypeStruct((_NB, 1, 128), jnp.float32),
        ],
        compiler_params=pltpu.CompilerParams(
            dimension_semantics=("arbitrary",),
        ),
    )(zf, c2col, maskf, cbm2)


def _sc_gather_body(cb_hbm, idx_hbm, out_hbm, idx_v, rows_v0, rows_v1,
                    sem0, sem1):
    wid = lax.axis_index("s") * _NC + lax.axis_index("c")
    base = wid * _B_PER_W
    rows_v = (rows_v0, rows_v1)
    sem = (sem0, sem1)
    pltpu.sync_copy(idx_hbm.at[pl.ds(base, _B_PER_W)], idx_v)
    descs = [None, None]
    descs[0] = pltpu.async_copy(cb_hbm.at[idx_v.at[pl.ds(0, _CH)]], rows_v[0],
                                sem[0])
    for c in range(_NCHUNK):
        cur, nxt = c % 2, (c + 1) % 2
        if c + 1 < _NCHUNK:
            descs[nxt] = pltpu.async_copy(
                cb_hbm.at[idx_v.at[pl.ds((c + 1) * _CH, _CH)]], rows_v[nxt],
                sem[nxt])
        descs[cur].wait()
        pltpu.sync_copy(rows_v[cur], out_hbm.at[pl.ds(base + c * _CH, _CH)])


@functools.lru_cache(maxsize=1)
def _sc_gather_kernel():
    return pl.kernel(
        _sc_gather_body,
        out_type=jax.ShapeDtypeStruct((_N, _D), jnp.float32),
        mesh=plsc.VectorSubcoreMesh(core_axis_name="c", subcore_axis_name="s",
                                    num_cores=_NC, num_subcores=_NS),
        scratch_types=[
            pltpu.VMEM((_B_PER_W,), jnp.int32),
            pltpu.VMEM((_CH, _D), jnp.float32),
            pltpu.VMEM((_CH, _D), jnp.float32),
            pltpu.SemaphoreType.DMA,
            pltpu.SemaphoreType.DMA,
        ],
    )


def kernel(z, mask, codebook):
    zf = z.reshape(_N, _D)
    c2col = jnp.sum(codebook * codebook, axis=-1).reshape(_K, 1)
    cbm2 = codebook * -2.0
    maskf = mask.astype(z.dtype).reshape(_NB, 1, _RB)
    idx3, pc, pn = _tc_argmin(zf, c2col, maskf, cbm2)
    idx = idx3.reshape(_N)
    quantized = _sc_gather_kernel()(codebook, idx)
    cnt = jnp.sum(pn[:, 0, 0])
    denom = jnp.maximum(cnt, 1.0) * _D
    commit_loss = jnp.sum(pc[:, 0, 0]) / denom
    return quantized.reshape(_B, _T, _D), idx.reshape(_B, _T), commit_loss


# speedup vs baseline: 1.4358x; 1.0011x over previous
import functools

import jax
import jax.numpy as jnp
from jax import lax
from jax.experimental import pallas as pl
from jax.experimental.pallas import tpu as pltpu
from jax.experimental.pallas import tpu_sc as plsc

_B, _T, _D, _K = 16, 2048, 256, 1024
_N = _B * _T
_RB = 1024
_NB = _N // _RB

_NC, _NS = 2, 16
_NW = _NC * _NS
_B_PER_W = _N // _NW
_CH = 128
_NCHUNK = _B_PER_W // _CH


def _tc_body(z_ref, c2_ref, m_ref, cbm2_ref, idx_ref, pc_ref, pn_ref):
    zb = z_ref[...]
    cbm2 = cbm2_ref[...]
    dots = lax.dot_general(cbm2, zb, (((1,), (1,)), ((), ())))
    c2 = c2_ref[...]
    z2col = jnp.sum(zb * zb, axis=1, keepdims=True)
    z2 = z2col.T
    dist = (z2 + dots) + c2
    minv = jnp.min(dist, axis=0, keepdims=True)
    kio = lax.broadcasted_iota(jnp.int32, (_K, 1), 0).astype(jnp.float32)
    idxf = jnp.min(jnp.where(dist == minv, kio, float(_K)), axis=0)
    idx = idxf.astype(jnp.int32)
    mrow = m_ref[0, 0, :]
    commit_p = jnp.sum(minv[0] * mrow)
    cnt_p = jnp.sum(mrow)
    idx_ref[0, 0, :] = idx
    pc_ref[0, 0, :] = jnp.full((128,), commit_p, jnp.float32)
    pn_ref[0, 0, :] = jnp.full((128,), cnt_p, jnp.float32)


def _tc_argmin(zf, c2col, maskf, cbm2):
    return pl.pallas_call(
        _tc_body,
        grid=(_NB,),
        in_specs=[
            pl.BlockSpec((_RB, _D), lambda i: (i, 0)),
            pl.BlockSpec((_K, 1), lambda i: (0, 0)),
            pl.BlockSpec((1, 1, _RB), lambda i: (i, 0, 0)),
            pl.BlockSpec((_K, _D), lambda i: (0, 0)),
        ],
        out_specs=[
            pl.BlockSpec((1, 1, _RB), lambda i: (i, 0, 0)),
            pl.BlockSpec((1, 1, 128), lambda i: (i, 0, 0)),
            pl.BlockSpec((1, 1, 128), lambda i: (i, 0, 0)),
        ],
        out_shape=[
            jax.ShapeDtypeStruct((_NB, 1, _RB), jnp.int32),
            jax.ShapeDtypeStruct((_NB, 1, 128), jnp.float32),
            jax.ShapeDtypeStruct((_NB, 1, 128), jnp.float32),
        ],
        compiler_params=pltpu.CompilerParams(
            dimension_semantics=("arbitrary",),
        ),
    )(zf, c2col, maskf, cbm2)


_NBUF = 3


def _sc_gather_body(cb_hbm, idx_hbm, out_hbm, idx_v, rows_v0, rows_v1,
                    rows_v2, gsem0, gsem1, gsem2, ssem0, ssem1, ssem2):
    wid = lax.axis_index("s") * _NC + lax.axis_index("c")
    base = wid * _B_PER_W
    rows_v = (rows_v0, rows_v1, rows_v2)
    gsem = (gsem0, gsem1, gsem2)
    ssem = (ssem0, ssem1, ssem2)
    pltpu.sync_copy(idx_hbm.at[pl.ds(base, _B_PER_W)], idx_v)
    gd = [None] * _NBUF
    sd = [None] * _NBUF
    for b in range(2):
        gd[b] = pltpu.async_copy(cb_hbm.at[idx_v.at[pl.ds(b * _CH, _CH)]],
                                 rows_v[b], gsem[b])
    for c in range(_NCHUNK):
        cur = c % _NBUF
        pre = c + 2
        if pre < _NCHUNK:
            pb = pre % _NBUF
            if sd[pb] is not None:
                sd[pb].wait()
            gd[pb] = pltpu.async_copy(
                cb_hbm.at[idx_v.at[pl.ds(pre * _CH, _CH)]], rows_v[pb],
                gsem[pb])
        gd[cur].wait()
        sd[cur] = pltpu.async_copy(rows_v[cur],
                                   out_hbm.at[pl.ds(base + c * _CH, _CH)],
                                   ssem[cur])
    for b in range(_NBUF):
        if sd[b] is not None:
            sd[b].wait()


@functools.lru_cache(maxsize=1)
def _sc_gather_kernel():
    return pl.kernel(
        _sc_gather_body,
        out_type=jax.ShapeDtypeStruct((_N, _D), jnp.float32),
        mesh=plsc.VectorSubcoreMesh(core_axis_name="c", subcore_axis_name="s",
                                    num_cores=_NC, num_subcores=_NS),
        scratch_types=[
            pltpu.VMEM((_B_PER_W,), jnp.int32),
            pltpu.VMEM((_CH, _D), jnp.float32),
            pltpu.VMEM((_CH, _D), jnp.float32),
            pltpu.VMEM((_CH, _D), jnp.float32),
            pltpu.SemaphoreType.DMA,
            pltpu.SemaphoreType.DMA,
            pltpu.SemaphoreType.DMA,
            pltpu.SemaphoreType.DMA,
            pltpu.SemaphoreType.DMA,
            pltpu.SemaphoreType.DMA,
        ],
    )


def kernel(z, mask, codebook):
    zf = z.reshape(_N, _D)
    c2col = jnp.sum(codebook * codebook, axis=-1).reshape(_K, 1)
    cbm2 = codebook * -2.0
    maskf = mask.astype(z.dtype).reshape(_NB, 1, _RB)
    idx3, pc, pn = _tc_argmin(zf, c2col, maskf, cbm2)
    idx = idx3.reshape(_N)
    quantized = _sc_gather_kernel()(codebook, idx)
    cnt = jnp.sum(pn[:, 0, 0])
    denom = jnp.maximum(cnt, 1.0) * _D
    commit_loss = jnp.sum(pc[:, 0, 0]) / denom
    return quantized.reshape(_B, _T, _D), idx.reshape(_B, _T), commit_loss
